# baseline (speedup 1.0000x reference)
import jax
import jax.numpy as jnp
from jax import lax
from jax.experimental import pallas as pl
from jax.experimental.pallas import tpu as pltpu
from jax.experimental.pallas import tpu_sc as plsc

B, S = 4096, 200
WORD_DIM = 128
POS_ROWS = 201
POS_SIZE = 16
OUT_DIM = WORD_DIM + 2 * POS_SIZE

NC, NS = 2, 16
NW = NC * NS
BW = B // NW
NITER = S
BLK = 8
NBLK = NITER // BLK
NSB = 12


def _emb_kernel(widx_hbm, p1idx_hbm, p2idx_hbm, wtab_hbm, p1tab_hbm, p2tab_hbm,
                out_hbm,
                widx_v0, widx_v1, p1idx_v0, p2idx_v0, p1idx_v1, p2idx_v1,
                wrows_v0, wrows_v1, outt_v0, outt_v1, p1tab_v, p2tab_v,
                sem_i0, sem_i1, sem_g0, sem_g1, sem_w0, sem_w1):
    wid = lax.axis_index("s") * NC + lax.axis_index("c")
    boff = wid * BW

    idxbufs = [(widx_v0, p1idx_v0, p2idx_v0, sem_i0),
               (widx_v1, p1idx_v1, p2idx_v1, sem_i1)]
    wrows = [(wrows_v0, sem_g0), (wrows_v1, sem_g1)]
    outts = [(outt_v0, sem_w0), (outt_v1, sem_w1)]

    def idxblk_copies(b, kb):
        widx_v, p1idx_v, p2idx_v, sem_i = idxbufs[kb]
        rows = pl.ds(b * BLK, BLK)
        cols = pl.ds(boff, BW)
        return [pltpu.make_async_copy(widx_hbm.at[rows, :, cols], widx_v,
                                      sem_i),
                pltpu.make_async_copy(p1idx_hbm.at[rows, cols], p1idx_v,
                                      sem_i),
                pltpu.make_async_copy(p2idx_hbm.at[rows, cols], p2idx_v,
                                      sem_i)]

    def gather_copy(j, k, kb):
        widx_v = idxbufs[kb][0]
        wrows_v, sem_g = wrows[k]
        return pltpu.make_async_copy(wtab_hbm.at[widx_v.at[j, 0]], wrows_v,
                                     sem_g)

    def wb_copy(s, k):
        outt_v, sem_w = outts[k]
        return pltpu.make_async_copy(
            outt_v, out_hbm.at[s, :, pl.ds(boff, BW)], sem_w)

    pltpu.sync_copy(p1tab_hbm, p1tab_v)
    pltpu.sync_copy(p2tab_hbm, p2tab_v)

    lane = lax.iota(jnp.int32, 16)
    rowvs = [bg * 16 + lane for bg in range(BW // 16)]

    def chunk_work(j, k, kb):
        p1idx_v, p2idx_v = idxbufs[kb][1], idxbufs[kb][2]
        outt_v = outts[k][0]
        wrows_v = wrows[k][0]

        @pl.loop(0, BW // 16)
        def _(bg):
            bsl = pl.ds(bg * 16, 16)
            pv1 = p1idx_v[j, bsl] * POS_SIZE
            pv2 = p2idx_v[j, bsl] * POS_SIZE

            @pl.loop(0, POS_SIZE)
            def _(c):
                outt_v[WORD_DIM + c, bsl] = plsc.load_gather(
                    p1tab_v, [pv1 + c])
                outt_v[WORD_DIM + POS_SIZE + c, bsl] = plsc.load_gather(
                    p2tab_v, [pv2 + c])

        gather_copy(j, k, kb).wait()

        @pl.loop(0, WORD_DIM)
        def _(d):
            dvec = jnp.full((16,), 0, jnp.int32) + d
            for bg in range(BW // 16):
                outt_v[d, pl.ds(bg * 16, 16)] = plsc.load_gather(
                    wrows_v, [rowvs[bg], dvec])

    for c in idxblk_copies(0, 0):
        c.start()
    for c in idxblk_copies(0, 0):
        c.wait()
    for c in idxblk_copies(1, 1):
        c.start()
    gather_copy(0, 0, 0).start()

    @pl.loop(0, NSB)
    def _(sb):
        for bi in range(2):
            for j in range(BLK):
                ci = bi * BLK + j
                k = ci % 2
                s = sb * (2 * BLK) + ci

                chunk_work(j, k, bi)

                if j == BLK - 1:
                    for c in idxblk_copies(sb * 2 + bi + 1, 1 - bi):
                        c.wait()

                    @pl.when(sb * 2 + bi + 2 < NBLK)
                    def _():
                        for c in idxblk_copies(sb * 2 + bi + 2, bi):
                            c.start()

                    gather_copy(0, 1 - k, 1 - bi).start()
                else:
                    gather_copy(j + 1, 1 - k, bi).start()

                if ci == 0:
                    @pl.when(sb >= 1)
                    def _():
                        wb_copy(s - 1, 1 - k).wait()
                else:
                    wb_copy(s - 1, 1 - k).wait()

                wb_copy(s, k).start()

    for j in range(BLK):
        s = NSB * (2 * BLK) + j
        k = s % 2
        chunk_work(j, k, 0)
        if j < BLK - 1:
            gather_copy(j + 1, 1 - k, 0).start()
        wb_copy(s - 1, 1 - k).wait()
        wb_copy(s, k).start()
    wb_copy(NITER - 1, (NITER - 1) % 2).wait()


@jax.jit
def _run(widxT, p1idxT, p2idxT, word_table, pos1_flat, pos2_flat):
    mesh = plsc.VectorSubcoreMesh(core_axis_name="c", subcore_axis_name="s",
                                  num_cores=NC, num_subcores=NS)
    return pl.kernel(
        _emb_kernel,
        out_type=jax.ShapeDtypeStruct((S, OUT_DIM, B), jnp.float32),
        mesh=mesh,
        compiler_params=pltpu.CompilerParams(needs_layout_passes=False,
                                             use_tc_tiling_on_sc=True),
        scratch_types=[
            pltpu.VMEM((BLK, 1, BW), jnp.int32),
            pltpu.VMEM((BLK, 1, BW), jnp.int32),
            pltpu.VMEM((BLK, BW), jnp.int32),
            pltpu.VMEM((BLK, BW), jnp.int32),
            pltpu.VMEM((BLK, BW), jnp.int32),
            pltpu.VMEM((BLK, BW), jnp.int32),
            pltpu.VMEM((BW, WORD_DIM), jnp.float32),
            pltpu.VMEM((BW, WORD_DIM), jnp.float32),
            pltpu.VMEM((OUT_DIM, BW), jnp.float32),
            pltpu.VMEM((OUT_DIM, BW), jnp.float32),
            pltpu.VMEM((POS_ROWS * POS_SIZE,), jnp.float32),
            pltpu.VMEM((POS_ROWS * POS_SIZE,), jnp.float32),
            pltpu.SemaphoreType.DMA,
            pltpu.SemaphoreType.DMA,
            pltpu.SemaphoreType.DMA,
            pltpu.SemaphoreType.DMA,
            pltpu.SemaphoreType.DMA,
            pltpu.SemaphoreType.DMA,
        ],
    )(widxT, p1idxT, p2idxT, word_table, pos1_flat, pos2_flat)


def kernel(input_word, input_pos1, input_pos2, word_table, pos1_table, pos2_table):
    widxT = input_word.T.reshape(S, 1, B).astype(jnp.int32)
    p1idxT = input_pos1.T.astype(jnp.int32)
    p2idxT = input_pos2.T.astype(jnp.int32)
    out = _run(widxT, p1idxT, p2idxT, word_table,
               pos1_table.reshape(-1), pos2_table.reshape(-1))
    return out.transpose(2, 0, 1)


# speedup vs baseline: 2.1692x; 2.1692x over previous
import jax
import jax.numpy as jnp
from jax import lax
from jax.experimental import pallas as pl
from jax.experimental.pallas import tpu as pltpu
from jax.experimental.pallas import tpu_sc as plsc

B, S = 4096, 200
WORD_DIM = 128
POS_ROWS = 201
POS_SIZE = 16
OUT_DIM = WORD_DIM + 2 * POS_SIZE

NC, NS = 2, 16
NW = NC * NS
N = B * S
PER_W = N // NW
CHUNK = 128
NITER = PER_W // CHUNK
BLK = 10
IBLK = BLK * CHUNK
NBLK = NITER // BLK
NSB = NBLK // 2


def _emb_kernel(widx_hbm, p1idx_hbm, p2idx_hbm, wtab_hbm, p1tab_hbm, p2tab_hbm,
                out_hbm,
                wsub_v0, wsub_v1,
                p1idx_v0, p2idx_v0, p1idx_v1, p2idx_v1,
                outbuf_v0, outbuf_v1, p1tab_v, p2tab_v,
                sem_i0, sem_i1, sem_l0, sem_l1, sem_g0, sem_g1, sem_w0,
                sem_w1):
    wid = lax.axis_index("s") * NC + lax.axis_index("c")
    base = wid * PER_W

    idxbufs = [(p1idx_v0, p2idx_v0, sem_i0), (p1idx_v1, p2idx_v1, sem_i1)]
    wsubs = [(wsub_v0, sem_l0), (wsub_v1, sem_l1)]
    outbufs = [(outbuf_v0, sem_g0, sem_w0), (outbuf_v1, sem_g1, sem_w1)]

    def idxblk_copies(b, kb):
        p1idx_v, p2idx_v, sem_i = idxbufs[kb]
        sl = pl.ds(base + b * IBLK, IBLK)
        return [pltpu.make_async_copy(p1idx_hbm.at[sl], p1idx_v, sem_i),
                pltpu.make_async_copy(p2idx_hbm.at[sl], p2idx_v, sem_i)]

    def widx_copy(i, kw):
        wsub_v, sem_l = wsubs[kw]
        return pltpu.make_async_copy(
            widx_hbm.at[pl.ds(base + i * CHUNK, CHUNK)], wsub_v, sem_l)

    def gather_copy(k, kw):
        wsub_v, _ = wsubs[kw]
        outbuf_v, sem_g, _ = outbufs[k]
        return pltpu.make_async_copy(
            wtab_hbm.at[wsub_v],
            outbuf_v.at[:, pl.ds(0, WORD_DIM)],
            sem_g)

    def wb_copy(i, k):
        outbuf_v, _, sem_w = outbufs[k]
        return pltpu.make_async_copy(
            outbuf_v, out_hbm.at[pl.ds(base + i * CHUNK, CHUNK)], sem_w)

    pltpu.sync_copy(p1tab_hbm, p1tab_v)
    pltpu.sync_copy(p2tab_hbm, p2tab_v)

    lane = lax.iota(jnp.int32, 16)

    for c in idxblk_copies(0, 0):
        c.start()
    widx_copy(0, 0).start()
    widx_copy(1, 1).start()
    for c in idxblk_copies(0, 0):
        c.wait()
    for c in idxblk_copies(1, 1):
        c.start()
    widx_copy(0, 0).wait()
    gather_copy(0, 0).start()

    @pl.loop(0, NSB)
    def _(sb):
        for bi in range(2):
            for j in range(BLK):
                ci = bi * BLK + j
                k = ci % 2
                i = sb * (2 * BLK) + ci
                p1idx_v, p2idx_v, _ = idxbufs[bi]
                outbuf_v = outbufs[k][0]
                ioff = j * CHUNK

                @pl.loop(0, CHUNK // 16)
                def _(gr):
                    rowv = gr * 16 + lane
                    pv1 = p1idx_v[pl.ds(ioff + gr * 16, 16)] * POS_SIZE
                    pv2 = p2idx_v[pl.ds(ioff + gr * 16, 16)] * POS_SIZE
                    for c in range(POS_SIZE):
                        v1 = plsc.load_gather(p1tab_v, [pv1 + c])
                        plsc.store_scatter(
                            outbuf_v,
                            [rowv, jnp.full((16,), WORD_DIM + c, jnp.int32)],
                            v1)
                        v2 = plsc.load_gather(p2tab_v, [pv2 + c])
                        plsc.store_scatter(
                            outbuf_v,
                            [rowv,
                             jnp.full((16,), WORD_DIM + POS_SIZE + c,
                                      jnp.int32)],
                            v2)

                if ci == 0:
                    @pl.when(sb >= 1)
                    def _():
                        wb_copy(i - 1, 1 - k).wait()
                else:
                    wb_copy(i - 1, 1 - k).wait()

                if j == BLK - 1:
                    last = (bi == 1)

                    def _boundary(nb_bi=1 - bi, sb_=sb, bi_=bi, k_=k):
                        for c in idxblk_copies(sb_ * 2 + bi_ + 1, nb_bi):
                            c.wait()

                        @pl.when(sb_ * 2 + bi_ + 2 < NBLK)
                        def _():
                            for c in idxblk_copies(sb_ * 2 + bi_ + 2, bi_):
                                c.start()

                        widx_copy(sb_ * (2 * BLK) + bi_ * BLK + BLK,
                                  1 - k_).wait()
                        gather_copy(1 - k_, 1 - k_).start()

                    if last:
                        @pl.when(sb < NSB - 1)
                        def _():
                            _boundary()
                    else:
                        _boundary()
                else:
                    widx_copy(i + 1, 1 - k).wait()
                    gather_copy(1 - k, 1 - k).start()

                gather_copy(k, k).wait()

                if ci >= 2 * BLK - 2:
                    @pl.when(sb < NSB - 1)
                    def _():
                        widx_copy(i + 2, k).start()
                else:
                    widx_copy(i + 2, k).start()

                wb_copy(i, k).start()

    wb_copy(NITER - 1, (NITER - 1) % 2).wait()


@jax.jit
def _run(widx, p1idx, p2idx, word_table, pos1_flat, pos2_flat):
    mesh = plsc.VectorSubcoreMesh(core_axis_name="c", subcore_axis_name="s",
                                  num_cores=NC, num_subcores=NS)
    return pl.kernel(
        _emb_kernel,
        out_type=jax.ShapeDtypeStruct((N, OUT_DIM), jnp.float32),
        mesh=mesh,
        compiler_params=pltpu.CompilerParams(needs_layout_passes=False,
                                             use_tc_tiling_on_sc=True),
        scratch_types=[
            pltpu.VMEM((CHUNK,), jnp.int32),
            pltpu.VMEM((CHUNK,), jnp.int32),
            pltpu.VMEM((IBLK,), jnp.int32),
            pltpu.VMEM((IBLK,), jnp.int32),
            pltpu.VMEM((IBLK,), jnp.int32),
            pltpu.VMEM((IBLK,), jnp.int32),
            pltpu.VMEM((CHUNK, OUT_DIM), jnp.float32),
            pltpu.VMEM((CHUNK, OUT_DIM), jnp.float32),
            pltpu.VMEM((POS_ROWS * POS_SIZE,), jnp.float32),
            pltpu.VMEM((POS_ROWS * POS_SIZE,), jnp.float32),
            pltpu.SemaphoreType.DMA,
            pltpu.SemaphoreType.DMA,
            pltpu.SemaphoreType.DMA,
            pltpu.SemaphoreType.DMA,
            pltpu.SemaphoreType.DMA,
            pltpu.SemaphoreType.DMA,
            pltpu.SemaphoreType.DMA,
            pltpu.SemaphoreType.DMA,
        ],
    )(widx, p1idx, p2idx, word_table, pos1_flat, pos2_flat)


def kernel(input_word, input_pos1, input_pos2, word_table, pos1_table, pos2_table):
    widx = input_word.reshape(-1).astype(jnp.int32)
    p1idx = input_pos1.reshape(-1).astype(jnp.int32)
    p2idx = input_pos2.reshape(-1).astype(jnp.int32)
    out = _run(widx, p1idx, p2idx, word_table,
               pos1_table.reshape(-1), pos2_table.reshape(-1))
    return out.reshape(B, S, OUT_DIM)
